# baseline (speedup 1.0000x reference)
import jax
import jax.numpy as jnp
from jax import lax
from jax.experimental import pallas as pl
from jax.experimental.pallas import tpu as pltpu

_B, _C, _S = 256, 768, 196
_NC = 10
_CH = 7
_NCK = _S // _CH
_RING = 4


def _body(f_hbm, w_ref, b_ref, o_ref, bufs, acc_ref, sems):
    def dma(r):
        slot = lax.rem(r, _RING)
        return pltpu.make_async_copy(
            f_hbm.at[pl.ds(r * _CH, _CH)], bufs.at[slot], sems.at[slot])

    for r in range(_RING):
        dma(r).start()

    def body(r, _):
        slot = lax.rem(r, _RING)
        dma(r).wait()
        partial = jnp.sum(bufs[slot], axis=0)

        @pl.when(r == 0)
        def _init():
            acc_ref[...] = partial

        @pl.when(r > 0)
        def _acc():
            acc_ref[...] += partial

        @pl.when(r + _RING < _NCK)
        def _next():
            dma(r + _RING).start()
        return 0

    lax.fori_loop(0, _NCK, body, 0)

    pooled = acc_ref[...] * (1.0 / _S)
    o_ref[...] = jax.lax.dot_general(
        pooled, w_ref[...], (((1,), (1,)), ((), ())),
        preferred_element_type=jnp.float32) + b_ref[...]


def kernel(features, W, b):
    f3 = features.transpose(2, 3, 0, 1).reshape(_S, _B, _C)
    out = pl.pallas_call(
        _body,
        in_specs=[
            pl.BlockSpec(memory_space=pltpu.HBM),
            pl.BlockSpec((_NC, _C), lambda: (0, 0)),
            pl.BlockSpec((1, _NC), lambda: (0, 0)),
        ],
        out_specs=pl.BlockSpec((_B, _NC), lambda: (0, 0)),
        out_shape=jax.ShapeDtypeStruct((_B, _NC), jnp.float32),
        scratch_shapes=[
            pltpu.VMEM((_RING, _CH, _B, _C), jnp.float32),
            pltpu.VMEM((_B, _C), jnp.float32),
            pltpu.SemaphoreType.DMA((_RING,)),
        ],
    )(f3, W, b.reshape(1, _NC))
    return out
